# baseline (speedup 1.0000x reference)
import functools
import math

import jax
import jax.numpy as jnp
from jax import lax
from jax.experimental import pallas as pl
from jax.experimental.pallas import tpu as pltpu
from jax.experimental.pallas import tpu_sc as plsc

EMB = 64
SCALE = math.sqrt(EMB)
SUB = 128
NBUF = 4
NOBUF = 2
LEAD = 2


def kernel(tokens, table):
    B, L = tokens.shape
    vocab, emb = table.shape
    assert emb == EMB and vocab % 2 == 0
    info = plsc.get_sparse_core_info()
    num_workers = info.num_cores * info.num_subcores
    bw = B // num_workers
    nsub = bw // SUB
    assert bw % SUB == 0 and nsub == NBUF
    steps = L * nsub

    tok_t = tokens.T.astype(jnp.int32)
    tab_p = table.reshape(vocab // 2, 2 * EMB)

    mesh = plsc.VectorSubcoreMesh(core_axis_name="c", subcore_axis_name="s")

    @functools.partial(
        pl.kernel,
        out_type=jax.ShapeDtypeStruct((L, EMB, B), jnp.float32),
        mesh=mesh,
        scratch_types=[
            pltpu.VMEM((L, bw), jnp.int32),
            [pltpu.VMEM((SUB,), jnp.int32) for _ in range(NBUF)],
            [pltpu.VMEM((SUB, 2 * EMB), jnp.float32) for _ in range(NBUF)],
            [pltpu.VMEM((EMB, SUB), jnp.float32) for _ in range(NOBUF)],
            [pltpu.SemaphoreType.DMA for _ in range(NBUF)],
            [pltpu.SemaphoreType.DMA for _ in range(NOBUF)],
        ],
        compiler_params=pltpu.CompilerParams(
            use_tc_tiling_on_sc=True, needs_layout_passes=False
        ),
    )
    def emb_lookup(tok_hbm, tab_hbm, out_hbm, tok_v, qidx, gbuf, obuf, gsem, wsem):
        wid = lax.axis_index("s") * info.num_cores + lax.axis_index("c")
        b0 = wid * bw
        iota = lax.iota(jnp.int32, 16)

        pltpu.sync_copy(tok_hbm.at[:, pl.ds(b0, bw)], tok_v)

        def prep_gather(l, s, gb):
            for g in range(SUB // 16):
                sl = pl.ds(g * 16, 16)
                t16 = tok_v[l, pl.ds(s * SUB + g * 16, 16)]
                qidx[gb][sl] = lax.shift_right_logical(t16, 1)
            pltpu.async_copy(tab_hbm.at[qidx[gb]], gbuf[gb], gsem[gb])

        def wait_gather(gb):
            pltpu.make_async_copy(tab_hbm.at[qidx[gb]], gbuf[gb], gsem[gb]).wait()

        def out_slice(l, s):
            return out_hbm.at[l, :, pl.ds(b0 + s * SUB, SUB)]

        def fire_wb(l, s, ob):
            pltpu.async_copy(obuf[ob], out_slice(l, s), wsem[ob])

        def wait_wb(l, s, ob):
            pltpu.make_async_copy(obuf[ob], out_slice(l, s), wsem[ob]).wait()

        def transpose_scale(l, s, gb, ob):
            def gloop(g, carry):
                t16 = tok_v[l, pl.ds(s * SUB + g * 16, 16)]
                rows = g * 16 + iota
                cols0 = (t16 & 1) * EMB

                def jloop(j, carry2):
                    vals = plsc.load_gather(gbuf[gb], [rows, cols0 + j])
                    obuf[ob][j, pl.ds(g * 16, 16)] = vals * SCALE
                    return carry2

                lax.fori_loop(0, EMB, jloop, 0, unroll=8)
                return carry

            lax.fori_loop(0, SUB // 16, gloop, 0)

        def refill(l, s, k):
            prep_gather(l, s, k % NBUF)

        for k in range(LEAD):
            refill(0, k, k)

        def outer(i, carry):
            l = i
            for j in range(nsub):
                k = j
                gb = j
                ob = j % NOBUF
                wait_gather(gb)
                jp = j - NOBUF
                lp = l + (jp // nsub)
                sp = jp % nsub
                wait_wb(lp, sp, ob)
                transpose_scale(l, j, gb, ob)
                fire_wb(l, j, ob)
                jn = j + LEAD
                ln = l + (jn // nsub)
                sn = jn % nsub
                refill(ln, sn, jn % NBUF)
            return carry

        l = 0
        for j in range(nsub):
            gb = j
            ob = j % NOBUF
            wait_gather(gb)
            if j >= NOBUF:
                wait_wb(0, j - NOBUF, ob)
            transpose_scale(0, j, gb, ob)
            fire_wb(0, j, ob)
            jn = j + LEAD
            refill((jn // nsub), jn % nsub, jn % NBUF)

        lax.fori_loop(1, L - 1, outer, 0)

        li = L - 1
        for j in range(nsub):
            gb = j
            ob = j % NOBUF
            wait_gather(gb)
            jp = j - NOBUF
            wait_wb(li + (jp // nsub), jp % nsub, ob)
            transpose_scale(li, j, gb, ob)
            fire_wb(li, j, ob)
            if j + LEAD < nsub:
                refill(li, j + LEAD, (j + LEAD) % NBUF)
        for j in range(nsub - NOBUF, nsub):
            wait_wb(li, j, j % NOBUF)

    out_t = emb_lookup(tok_t, tab_p)
    return out_t.transpose(2, 0, 1)
